# baseline (speedup 1.0000x reference)
import functools

import jax
import jax.numpy as jnp
from jax import lax
from jax.experimental import pallas as pl
from jax.experimental.pallas import tpu as pltpu
from jax.experimental.pallas import tpu_sc as plsc

VOCAB = 1000000
EMBED = 64
BATCH = 16384
NEG_K = 20

NC = 2
NS = 16
NW = NC * NS

B_W = BATCH // NW
PAIRS = 128
WJOB = 2 * PAIRS
NKP = NEG_K // 2
NBLK = B_W // PAIRS
NJN = NKP * NBLK

_mesh = plsc.VectorSubcoreMesh(
    core_axis_name="c", subcore_axis_name="s", num_cores=NC, num_subcores=NS
)

_PIB = "wrap"


@functools.partial(
    pl.kernel,
    out_type=(
        jax.ShapeDtypeStruct((BATCH // 2, 128), jnp.float32),
        jax.ShapeDtypeStruct((BATCH // 2, 128), jnp.float32),
        jax.ShapeDtypeStruct((BATCH, NKP * 128), jnp.float32),
    ),
    mesh=_mesh,
    scratch_types=[
        pltpu.VMEM((B_W,), jnp.int32),
        pltpu.VMEM((B_W,), jnp.int32),
        pltpu.VMEM((NEG_K, B_W), jnp.int32),
        pltpu.VMEM((WJOB,), jnp.int32),
        pltpu.VMEM((WJOB,), jnp.int32),
        pltpu.VMEM((WJOB, 128), jnp.float32),
        pltpu.VMEM((WJOB, 128), jnp.float32),
        pltpu.VMEM((PAIRS, 128), jnp.float32),
        pltpu.VMEM((PAIRS, 128), jnp.float32),
        pltpu.SemaphoreType.DMA,
        pltpu.SemaphoreType.DMA,
        pltpu.SemaphoreType.DMA,
    ],
    compiler_params=pltpu.CompilerParams(use_tc_tiling_on_sc=True),
)
def _sc_gather(center_hbm, pos_hbm, negt_hbm, in2_hbm, out2_hbm,
               oc, op, on,
               idxc, idxp, idxn, g0, g1, ch0, ch1, hb0, hb1,
               semi, sem0, sem1):
    wid = lax.axis_index("s") * NC + lax.axis_index("c")
    base = pl.multiple_of(wid * B_W, B_W)
    lanes = lax.iota(jnp.int32, 16)
    take_lo = lax.rem(lanes, 8) * 2
    take_hi = take_lo + 1

    dc = pltpu.async_copy(center_hbm.at[pl.ds(base, B_W)], idxc, semi)
    dp = pltpu.async_copy(pos_hbm.at[pl.ds(base, B_W)], idxp, semi)
    dn = pltpu.async_copy(negt_hbm.at[:, pl.ds(base, B_W)], idxn, semi)

    def shift_into(src_load, gbuf, goff, n16):
        def body(i, _):
            o = pl.multiple_of(i * 16, 16)
            gbuf[pl.ds(goff + o, 16)] = lax.shift_right_logical(src_load(o), 1)
            return ()
        lax.fori_loop(0, n16, body, ())

    def take(v, idx16):
        return jnp.take(v, idx16, mode=_PIB)

    def splat(v, j):
        return take(v, jnp.full((16,), 0, jnp.int32) + j)

    def select_pairs(chunk, hbuf, h_a, h_b, r_a, r_b):
        def grp(g, _):
            ha16 = h_a(g)
            hb16 = h_b(g)
            def pair(q, _):
                p = g * 16 + q
                fa = splat(ha16, q).astype(jnp.float32)
                fb = splat(hb16, q).astype(jnp.float32)
                ra = r_a(p)
                rb = r_b(p)
                for k in range(4):
                    ko = 16 * k
                    alo = chunk[ra, pl.ds(ko, 16)]
                    ahi = chunk[ra, pl.ds(64 + ko, 16)]
                    hbuf[p, pl.ds(ko, 16)] = alo + (ahi - alo) * fa
                    blo = chunk[rb, pl.ds(ko, 16)]
                    bhi = chunk[rb, pl.ds(64 + ko, 16)]
                    hbuf[p, pl.ds(64 + ko, 16)] = blo + (bhi - blo) * fb
                return ()
            lax.fori_loop(0, 16, pair, ())
            return ()
        lax.fori_loop(0, PAIRS // 16, grp, ())

    bufs = ((g0, ch0, hb0, sem0), (g1, ch1, hb1, sem1))

    dc.wait()
    dp.wait()
    for src_idx, table, out in ((idxc, in2_hbm, oc), (idxp, out2_hbm, op)):
        descs = []
        for c in range(2):
            g, ch, hb, sem = bufs[c]
            shift_into(lambda o, c=c, s=src_idx: s[pl.ds(c * WJOB + o, 16)],
                       g, 0, WJOB // 16)
            descs.append(pltpu.async_copy(table.at[g], ch, sem))
        for c in range(2):
            g, ch, hb, sem = bufs[c]
            descs[c].wait()
            def h_ab(gi, c=c, s=src_idx, hi=False):
                w0 = s[pl.ds(c * WJOB + 32 * gi, 16)]
                w1 = s[pl.ds(c * WJOB + 32 * gi + 16, 16)]
                tk = take_hi if hi else take_lo
                m = lax.shift_right_logical(lanes, 3)
                return lax.bitwise_and(
                    take(w0, tk) * (1 - m) + take(w1, tk) * m, 1)
            select_pairs(ch, hb,
                         functools.partial(h_ab, hi=False),
                         functools.partial(h_ab, hi=True),
                         lambda p: 2 * p, lambda p: 2 * p + 1)
            ob = pl.multiple_of(wid * (B_W // 2), B_W // 2)
            pltpu.sync_copy(hb, out.at[pl.ds(ob + c * PAIRS, PAIRS)])

    dn.wait()

    def neg_prep(j, parity):
        g, ch, hb, sem = bufs[parity]
        kp = j // NBLK
        blk = lax.rem(j, NBLK)
        bo = pl.multiple_of(blk * PAIRS, PAIRS)
        shift_into(lambda o, r=2 * kp: idxn[r, pl.ds(bo + o, 16)],
                   g, 0, PAIRS // 16)
        shift_into(lambda o, r=2 * kp + 1: idxn[r, pl.ds(bo + o, 16)],
                   g, PAIRS, PAIRS // 16)
        d0 = pltpu.async_copy(out2_hbm.at[g.at[pl.ds(0, PAIRS)]],
                              ch.at[pl.ds(0, PAIRS)], sem)
        d1 = pltpu.async_copy(out2_hbm.at[g.at[pl.ds(PAIRS, PAIRS)]],
                              ch.at[pl.ds(PAIRS, PAIRS)], sem)
        return d0, d1

    def neg_finish(j, parity):
        g, ch, hb, sem = bufs[parity]
        kp = j // NBLK
        blk = lax.rem(j, NBLK)
        bo = pl.multiple_of(blk * PAIRS, PAIRS)
        pltpu.make_async_copy(out2_hbm.at[pl.ds(0, PAIRS)],
                              ch.at[pl.ds(0, PAIRS)], sem).wait()
        pltpu.make_async_copy(out2_hbm.at[pl.ds(0, PAIRS)],
                              ch.at[pl.ds(PAIRS, PAIRS)], sem).wait()

        def h_of(gi, row0):
            return lax.bitwise_and(idxn[row0, pl.ds(bo + 16 * gi, 16)], 1)

        select_pairs(ch, hb,
                     lambda gi: h_of(gi, 2 * kp),
                     lambda gi: h_of(gi, 2 * kp + 1),
                     lambda p: p, lambda p: PAIRS + p)
        col = pl.multiple_of(kp * 128, 128)
        pltpu.sync_copy(hb, on.at[pl.ds(base + bo, PAIRS), pl.ds(col, 128)])

    neg_prep(0, 0)
    neg_prep(1, 1)

    def loop(j, _):
        neg_finish(j, lax.rem(j, 2))
        return ()

    def loop2(j, _):
        ja = 2 * j
        neg_finish(ja, 0)
        neg_prep(ja + 2, 0)
        neg_finish(ja + 1, 1)
        neg_prep(ja + 3, 1)
        return ()
    lax.fori_loop(0, (NJN - 2) // 2, loop2, ())
    neg_finish(NJN - 2, 0)
    neg_finish(NJN - 1, 1)


@jax.jit
def kernel(center_word, pos_word, neg_word, in_embed, out_embed):
    in2 = in_embed.reshape(VOCAB // 2, 128)
    out2 = out_embed.reshape(VOCAB // 2, 128)
    oc, op, on = _sc_gather(center_word, pos_word, neg_word.T, in2, out2)
    return (oc.reshape(BATCH, EMBED), op.reshape(BATCH, EMBED),
            on.reshape(BATCH, NEG_K, EMBED))


# speedup vs baseline: 1.0070x; 1.0070x over previous
import functools

import jax
import jax.numpy as jnp
from jax import lax
from jax.experimental import pallas as pl
from jax.experimental.pallas import tpu as pltpu
from jax.experimental.pallas import tpu_sc as plsc

VOCAB = 1000000
EMBED = 64
BATCH = 16384
NEG_K = 20

NC = 2
NS = 16
NW = NC * NS

B_W = BATCH // NW
PAIRS = 128
WJOB = 2 * PAIRS
NKP = NEG_K // 2
NBLK = B_W // PAIRS
NJN = NKP * NBLK

_mesh = plsc.VectorSubcoreMesh(
    core_axis_name="c", subcore_axis_name="s", num_cores=NC, num_subcores=NS
)


@functools.partial(
    pl.kernel,
    out_type=(
        jax.ShapeDtypeStruct((BATCH // 2, 128), jnp.float32),
        jax.ShapeDtypeStruct((BATCH // 2, 128), jnp.float32),
        jax.ShapeDtypeStruct((BATCH, NKP * 128), jnp.float32),
    ),
    mesh=_mesh,
    scratch_types=[
        pltpu.VMEM((B_W,), jnp.int32),
        pltpu.VMEM((B_W,), jnp.int32),
        pltpu.VMEM((NEG_K, B_W), jnp.int32),
        pltpu.VMEM((WJOB,), jnp.int32),
        pltpu.VMEM((WJOB,), jnp.int32),
        pltpu.VMEM((WJOB, 128), jnp.float32),
        pltpu.VMEM((WJOB, 128), jnp.float32),
        pltpu.VMEM((PAIRS, 128), jnp.float32),
        pltpu.VMEM((PAIRS, 128), jnp.float32),
        pltpu.SemaphoreType.DMA,
        pltpu.SemaphoreType.DMA,
        pltpu.SemaphoreType.DMA,
    ],
    compiler_params=pltpu.CompilerParams(use_tc_tiling_on_sc=True),
)
def _sc_gather(center_hbm, pos_hbm, negt_hbm, in2_hbm, out2_hbm,
               oc, op, on,
               idxc, idxp, idxn, g0, g1, ch0, ch1, hb0, hb1,
               semi, sem0, sem1):
    wid = lax.axis_index("s") * NC + lax.axis_index("c")
    base = pl.multiple_of(wid * B_W, B_W)
    lanes = lax.iota(jnp.int32, 16)
    take_lo = lax.rem(lanes, 8) * 2
    take_hi = take_lo + 1

    dc = pltpu.async_copy(center_hbm.at[pl.ds(base, B_W)], idxc, semi)
    dp = pltpu.async_copy(pos_hbm.at[pl.ds(base, B_W)], idxp, semi)
    dn = pltpu.async_copy(negt_hbm.at[:, pl.ds(base, B_W)], idxn, semi)

    def shift_into(src_load, gbuf, goff, n16):
        def body(i, _):
            o = pl.multiple_of(i * 16, 16)
            gbuf[pl.ds(goff + o, 16)] = lax.shift_right_logical(src_load(o), 1)
            return ()
        lax.fori_loop(0, n16, body, ())

    def take(v, idx16):
        return jnp.take(v, idx16, mode="wrap")

    def splat(v, j):
        return take(v, jnp.full((16,), 0, jnp.int32) + j)

    def select_pairs(chunk, hbuf, h_a, h_b, r_a, r_b):
        def grp(g, _):
            ha16 = h_a(g)
            hb16 = h_b(g)

            def quad(t, _):
                for u in range(4):
                    q = t * 4 + u
                    p = g * 16 + q
                    fa = splat(ha16, q).astype(jnp.float32)
                    fb = splat(hb16, q).astype(jnp.float32)
                    ra = r_a(p)
                    rb = r_b(p)
                    for k in range(4):
                        ko = 16 * k
                        alo = chunk[ra, pl.ds(ko, 16)]
                        ahi = chunk[ra, pl.ds(64 + ko, 16)]
                        hbuf[p, pl.ds(ko, 16)] = alo + (ahi - alo) * fa
                        blo = chunk[rb, pl.ds(ko, 16)]
                        bhi = chunk[rb, pl.ds(64 + ko, 16)]
                        hbuf[p, pl.ds(64 + ko, 16)] = blo + (bhi - blo) * fb
                return ()
            lax.fori_loop(0, 4, quad, ())
            return ()
        lax.fori_loop(0, PAIRS // 16, grp, ())

    bufs = ((g0, ch0, hb0, sem0), (g1, ch1, hb1, sem1))

    dc.wait()
    dp.wait()
    for src_idx, table, out in ((idxc, in2_hbm, oc), (idxp, out2_hbm, op)):
        descs = []
        for c in range(2):
            g, ch, hb, sem = bufs[c]
            shift_into(lambda o, c=c, s=src_idx: s[pl.ds(c * WJOB + o, 16)],
                       g, 0, WJOB // 16)
            descs.append(pltpu.async_copy(table.at[g], ch, sem))
        for c in range(2):
            g, ch, hb, sem = bufs[c]
            descs[c].wait()

            def h_ab(gi, c=c, s=src_idx, hi=False):
                w0 = s[pl.ds(c * WJOB + 32 * gi, 16)]
                w1 = s[pl.ds(c * WJOB + 32 * gi + 16, 16)]
                tk = take_hi if hi else take_lo
                m = lax.shift_right_logical(lanes, 3)
                return lax.bitwise_and(
                    take(w0, tk) * (1 - m) + take(w1, tk) * m, 1)

            select_pairs(ch, hb,
                         functools.partial(h_ab, hi=False),
                         functools.partial(h_ab, hi=True),
                         lambda p: 2 * p, lambda p: 2 * p + 1)
            ob = pl.multiple_of(wid * (B_W // 2), B_W // 2)
            pltpu.sync_copy(hb, out.at[pl.ds(ob + c * PAIRS, PAIRS)])

    dn.wait()

    def neg_prep(j, parity):
        g, ch, hb, sem = bufs[parity]
        kp = j // NBLK
        blk = lax.rem(j, NBLK)
        bo = pl.multiple_of(blk * PAIRS, PAIRS)
        shift_into(lambda o, r=2 * kp: idxn[r, pl.ds(bo + o, 16)],
                   g, 0, PAIRS // 16)
        shift_into(lambda o, r=2 * kp + 1: idxn[r, pl.ds(bo + o, 16)],
                   g, PAIRS, PAIRS // 16)
        pltpu.async_copy(out2_hbm.at[g.at[pl.ds(0, PAIRS)]],
                         ch.at[pl.ds(0, PAIRS)], sem)
        pltpu.async_copy(out2_hbm.at[g.at[pl.ds(PAIRS, PAIRS)]],
                         ch.at[pl.ds(PAIRS, PAIRS)], sem)

    def neg_finish(j, parity):
        g, ch, hb, sem = bufs[parity]
        kp = j // NBLK
        blk = lax.rem(j, NBLK)
        bo = pl.multiple_of(blk * PAIRS, PAIRS)
        pltpu.make_async_copy(out2_hbm.at[pl.ds(0, PAIRS)],
                              ch.at[pl.ds(0, PAIRS)], sem).wait()
        pltpu.make_async_copy(out2_hbm.at[pl.ds(0, PAIRS)],
                              ch.at[pl.ds(PAIRS, PAIRS)], sem).wait()

        def h_of(gi, row0):
            return lax.bitwise_and(idxn[row0, pl.ds(bo + 16 * gi, 16)], 1)

        select_pairs(ch, hb,
                     lambda gi: h_of(gi, 2 * kp),
                     lambda gi: h_of(gi, 2 * kp + 1),
                     lambda p: p, lambda p: PAIRS + p)
        col = pl.multiple_of(kp * 128, 128)
        pltpu.sync_copy(hb, on.at[pl.ds(base + bo, PAIRS), pl.ds(col, 128)])

    neg_prep(0, 0)
    neg_prep(1, 1)

    def loop2(j, _):
        ja = 2 * j
        neg_finish(ja, 0)
        neg_prep(ja + 2, 0)
        neg_finish(ja + 1, 1)
        neg_prep(ja + 3, 1)
        return ()
    lax.fori_loop(0, (NJN - 2) // 2, loop2, ())
    neg_finish(NJN - 2, 0)
    neg_finish(NJN - 1, 1)


@jax.jit
def kernel(center_word, pos_word, neg_word, in_embed, out_embed):
    in_flat = lax.optimization_barrier(in_embed.reshape(VOCAB * EMBED))
    out_flat = lax.optimization_barrier(out_embed.reshape(VOCAB * EMBED))
    in2 = in_flat.reshape(VOCAB // 2, 128)
    out2 = out_flat.reshape(VOCAB // 2, 128)
    oc, op, on = _sc_gather(center_word, pos_word, neg_word.T, in2, out2)
    return (oc.reshape(BATCH, EMBED), op.reshape(BATCH, EMBED),
            on.reshape(BATCH, NEG_K, EMBED))


# speedup vs baseline: 1.2112x; 1.2027x over previous
import functools

import jax
import jax.numpy as jnp
from jax import lax
from jax.experimental import pallas as pl
from jax.experimental.pallas import tpu as pltpu
from jax.experimental.pallas import tpu_sc as plsc

VOCAB = 1000000
EMBED = 64
BATCH = 16384
NEG_K = 20

NC = 2
NS = 16
NW = NC * NS

B_W = BATCH // NW

_mesh = plsc.VectorSubcoreMesh(
    core_axis_name="c", subcore_axis_name="s", num_cores=NC, num_subcores=NS
)


@functools.partial(
    pl.kernel,
    out_type=(
        jax.ShapeDtypeStruct((BATCH, EMBED), jnp.float32),
        jax.ShapeDtypeStruct((BATCH, EMBED), jnp.float32),
        jax.ShapeDtypeStruct((BATCH, NEG_K * EMBED), jnp.float32),
    ),
    mesh=_mesh,
    scratch_types=[
        pltpu.VMEM((B_W,), jnp.int32),
        pltpu.VMEM((B_W,), jnp.int32),
        pltpu.VMEM((NEG_K, B_W), jnp.int32),
        pltpu.VMEM((B_W, EMBED), jnp.float32),
        pltpu.VMEM((B_W, EMBED), jnp.float32),
        pltpu.SemaphoreType.DMA,
        pltpu.SemaphoreType.DMA,
        pltpu.SemaphoreType.DMA,
    ],
    compiler_params=pltpu.CompilerParams(use_tc_tiling_on_sc=False),
)
def _sc_gather(center_hbm, pos_hbm, negt_hbm, in_hbm, out_hbm,
               o_center, o_pos, o_neg,
               idxc, idxp, idxn, bufa, bufb, semi, sema, semb):
    wid = lax.axis_index("s") * NC + lax.axis_index("c")
    base = pl.multiple_of(wid * B_W, B_W)

    di = pltpu.async_copy(center_hbm.at[pl.ds(base, B_W)], idxc, semi)
    dp = pltpu.async_copy(pos_hbm.at[pl.ds(base, B_W)], idxp, semi)
    dn = pltpu.async_copy(negt_hbm.at[:, pl.ds(base, B_W)], idxn, semi)
    di.wait()
    ga = pltpu.async_copy(in_hbm.at[idxc], bufa, sema)
    dp.wait()
    gb = pltpu.async_copy(out_hbm.at[idxp], bufb, semb)
    ga.wait()
    pltpu.sync_copy(bufa, o_center.at[pl.ds(base, B_W)])
    dn.wait()
    pltpu.async_copy(out_hbm.at[idxn.at[0]], bufa, sema)
    gb.wait()
    pltpu.sync_copy(bufb, o_pos.at[pl.ds(base, B_W)])
    pltpu.async_copy(out_hbm.at[idxn.at[1]], bufb, semb)

    def neg_out(k):
        return o_neg.at[pl.ds(base, B_W),
                        pl.ds(pl.multiple_of(k * EMBED, EMBED), EMBED)]

    def neg_pair(j, _):
        ka = j * 2
        pltpu.make_async_copy(out_hbm.at[pl.ds(0, B_W)], bufa, sema).wait()
        pltpu.sync_copy(bufa, neg_out(ka))
        pltpu.async_copy(out_hbm.at[idxn.at[ka + 2]], bufa, sema)
        pltpu.make_async_copy(out_hbm.at[pl.ds(0, B_W)], bufb, semb).wait()
        pltpu.sync_copy(bufb, neg_out(ka + 1))
        pltpu.async_copy(out_hbm.at[idxn.at[ka + 3]], bufb, semb)
        return ()
    lax.fori_loop(0, (NEG_K - 2) // 2, neg_pair, ())

    pltpu.make_async_copy(out_hbm.at[pl.ds(0, B_W)], bufa, sema).wait()
    pltpu.sync_copy(bufa, neg_out(NEG_K - 2))
    pltpu.make_async_copy(out_hbm.at[pl.ds(0, B_W)], bufb, semb).wait()
    pltpu.sync_copy(bufb, neg_out(NEG_K - 1))


@jax.jit
def kernel(center_word, pos_word, neg_word, in_embed, out_embed):
    emb, pos, neg = _sc_gather(center_word, pos_word, neg_word.T,
                               in_embed, out_embed)
    return emb, pos, neg.reshape(BATCH, NEG_K, EMBED)
